# baseline (speedup 1.0000x reference)
import functools

import jax
import jax.numpy as jnp
from jax import lax
from jax.experimental import pallas as pl
from jax.experimental.pallas import tpu as pltpu
from jax.experimental.pallas import tpu_sc as plsc

N = 10000
E = 320000
D = 128
H = 256
C = 8
B = 64

N2 = 10240
CH = 128
NB = 2
BLK = 10
NBT = 4
BLKT = 8
NC = 2
NS = 16
EPAD = 327680
NROW = EPAD // CH
RPT = N2 // NS
NCH1 = NROW // (NC * NS)
NCHZ = NROW // NS
TN = 512
NT = N2 // TN

@functools.lru_cache(maxsize=None)
def _mesh():
    return plsc.VectorSubcoreMesh(core_axis_name="c", subcore_axis_name="s",
                                  num_cores=NC, num_subcores=NS)


def _idx_block(src_hbm, dst_hbm, sed, ded, st, blkrow, blk, coff):
    pltpu.sync_copy(src_hbm.at[pl.ds(blkrow, blk)], sed.at[st])
    pltpu.sync_copy(dst_hbm.at[pl.ds(blkrow, blk)], ded.at[st])
    if coff is not None:
        for j in range(blk):
            for k in range(CH // 16):
                sl = pl.ds(k * 16, 16)
                sed[st, j, sl] = sed[st, j, sl] + coff


def _ring(table, src_hbm, dst_hbm, sed, ded, rows, gsems, ssems, acc,
          nb, ng, blk, base, nch, coff=None):
    nblk = nch // blk

    def ref(st, j, k):
        return (st, j - k) if j >= k else (1 - st, blk + j - k)

    def finalize(st1, j1, b):
        pltpu.make_async_copy(table.at[sed.at[st1, j1]], rows.at[b],
                              gsems[b]).wait()
        pltpu.async_copy(rows.at[b], acc.at[ded.at[st1, j1]], ssems[b],
                         add=True)

    def swait(st2, j2, b):
        pltpu.make_async_copy(rows.at[b], acc.at[ded.at[st2, j2]],
                              ssems[b]).wait()

    def block(st, blkrow, first):
        _idx_block(src_hbm, dst_hbm, sed, ded, st, blkrow, blk, coff)
        for j in range(blk):
            b = j % nb
            if not (first and j < ng):
                st1, j1 = ref(st, j, ng)
                finalize(st1, j1, (j - ng) % nb)
            if not (first and j < nb):
                st2, j2 = ref(st, j, nb)
                swait(st2, j2, b)
            pltpu.async_copy(table.at[sed.at[st, j]], rows.at[b], gsems[b])

    block(0, base, True)
    block(1, base + blk, False)

    def pair(p, carry):
        row0 = base + 2 * p * blk
        block(0, row0, False)
        block(1, row0 + blk, False)
        return carry

    lax.fori_loop(1, nblk // 2, pair, 0)
    for j in range(blk, blk + ng):
        finalize(1, j - ng, (j - ng) % nb)
    for j in range(blk - nb, blk):
        swait(1, j, j % nb)


def _sca_body(dst_hbm, zdeg_hbm, ones_hbm, degp_hbm, accd, ded, ones_v,
              sem0, sem1):
    c = lax.axis_index("c")
    s = lax.axis_index("s")
    sems = (sem0, sem1)
    pltpu.sync_copy(zdeg_hbm, accd.at[pl.ds(s * RPT, RPT)])
    pltpu.sync_copy(ones_hbm, ones_v)
    plsc.subcore_barrier()

    base = (c * NS + s) * NCH1
    nblk = NCH1 // BLK

    def swait(st, j, b):
        pltpu.make_async_copy(ones_v, accd.at[ded.at[st, j]], sems[b]).wait()

    def block(st, blkrow, first):
        pltpu.sync_copy(dst_hbm.at[pl.ds(blkrow, BLK)], ded.at[st])
        for j in range(BLK):
            b = j % NB
            if first and j < NB:
                pass
            elif j < NB:
                swait(1 - st, BLK - NB + j, b)
            else:
                swait(st, j - NB, b)
            pltpu.async_copy(ones_v, accd.at[ded.at[st, j]], sems[b],
                             add=True)

    block(0, base, True)
    block(1, base + BLK, False)

    def pair(p, carry):
        row0 = base + 2 * p * BLK
        block(0, row0, False)
        block(1, row0 + BLK, False)
        return carry

    lax.fori_loop(1, nblk // 2, pair, 0)
    for j in range(BLK - NB, BLK):
        swait(1, j, j % NB)

    plsc.subcore_barrier()
    pltpu.sync_copy(accd.at[pl.ds(s * RPT, RPT)],
                    degp_hbm.at[c, pl.ds(s * RPT, RPT)])


@functools.lru_cache(maxsize=None)
def _make_sca():
    return pl.kernel(
        _sca_body,
        out_type=jax.ShapeDtypeStruct((NC, N2, 16), jnp.float32),
        mesh=_mesh(),
        compiler_params=pltpu.CompilerParams(use_tc_tiling_on_sc=False),
        scratch_types=[
            pltpu.VMEM_SHARED((N2, 16), jnp.float32),
            pltpu.VMEM((2, BLK, CH), jnp.int32),
            pltpu.VMEM((CH, 16), jnp.float32),
            pltpu.SemaphoreType.DMA,
            pltpu.SemaphoreType.DMA,
        ],
    )


def _sca(*args):
    return _make_sca()(*args)


def _scw_body(table_hbm, src_hbm, dst_hbm, zrow_hbm, agg_hbm,
              acc, sed, ded, rows, gsem0, gsem1, ssem0, ssem1,
              *, split32, use_coff):
    c = lax.axis_index("c")
    s = lax.axis_index("s")
    pltpu.sync_copy(zrow_hbm, acc.at[pl.ds(s * RPT, RPT)])
    plsc.subcore_barrier()

    if split32:
        base = (c * NS + s) * NCH1
        nch = NCH1
    else:
        base = s * NCHZ
        nch = NCHZ
    _ring(table_hbm, src_hbm, dst_hbm, sed, ded, rows, (gsem0, gsem1),
          (ssem0, ssem1), acc, NB, 1, BLK, base, nch,
          coff=(c * N2 if use_coff else None))

    plsc.subcore_barrier()
    pltpu.sync_copy(acc.at[pl.ds(s * RPT, RPT)],
                    agg_hbm.at[c, pl.ds(s * RPT, RPT)])


@functools.lru_cache(maxsize=None)
def _make_scw(split32, use_coff):
    body = functools.partial(_scw_body, split32=split32, use_coff=use_coff)
    return pl.kernel(
        body,
        out_type=jax.ShapeDtypeStruct((NC, N2, D), jnp.float32),
        mesh=_mesh(),
        compiler_params=pltpu.CompilerParams(use_tc_tiling_on_sc=False),
        scratch_types=[
            pltpu.VMEM_SHARED((N2, D), jnp.float32),
            pltpu.VMEM((2, BLK, CH), jnp.int32),
            pltpu.VMEM((2, BLK, CH), jnp.int32),
            pltpu.VMEM((NB, CH, D), jnp.float32),
            pltpu.SemaphoreType.DMA,
            pltpu.SemaphoreType.DMA,
            pltpu.SemaphoreType.DMA,
            pltpu.SemaphoreType.DMA,
        ],
    )


def _scb(x2, srcR, dstR, zrow):
    return _make_scw(True, False)(x2, srcR, dstR, zrow)


def _scc(z1f, srcR, dstR, zrow):
    return _make_scw(False, True)(z1f, srcR, dstR, zrow)


def _scd_body(t16_hbm, src_hbm, dst_hbm, zdeg_hbm, aggt_hbm,
              acct, sed, ded, trows, g0, g1, g2, g3, s0, s1, s2, s3):
    c = lax.axis_index("c")
    s = lax.axis_index("s")
    pltpu.sync_copy(zdeg_hbm, acct.at[pl.ds(s * RPT, RPT)])
    plsc.subcore_barrier()

    _ring(t16_hbm, src_hbm, dst_hbm, sed, ded, trows, (g0, g1, g2, g3),
          (s0, s1, s2, s3), acct, NBT, 2, BLKT, (c * NS + s) * NCH1, NCH1)

    plsc.subcore_barrier()
    pltpu.sync_copy(acct.at[pl.ds(s * RPT, RPT)],
                    aggt_hbm.at[c, pl.ds(s * RPT, RPT)])


@functools.lru_cache(maxsize=None)
def _make_scd():
    return pl.kernel(
        _scd_body,
        out_type=jax.ShapeDtypeStruct((NC, N2, 16), jnp.float32),
        mesh=_mesh(),
        compiler_params=pltpu.CompilerParams(use_tc_tiling_on_sc=False),
        scratch_types=[
            pltpu.VMEM_SHARED((N2, 16), jnp.float32),
            pltpu.VMEM((2, BLKT, CH), jnp.int32),
            pltpu.VMEM((2, BLKT, CH), jnp.int32),
            pltpu.VMEM((NBT, CH, 16), jnp.float32),
            pltpu.SemaphoreType.DMA,
            pltpu.SemaphoreType.DMA,
            pltpu.SemaphoreType.DMA,
            pltpu.SemaphoreType.DMA,
            pltpu.SemaphoreType.DMA,
            pltpu.SemaphoreType.DMA,
            pltpu.SemaphoreType.DMA,
            pltpu.SemaphoreType.DMA,
        ],
    )


def _scd(*args):
    return _make_scd()(*args)


def _tc1_body(x_ref, aggx_ref, degp_ref,
              we1r_ref, we1n_ref, we1b_ref, wa1r_ref, wa1n_ref, wa1b_ref,
              wa2r_ref, wa2n_ref,
              z1_ref, t16_ref, r8_ref, invd_ref):
    hp = jax.lax.Precision.HIGHEST
    xt = x_ref[...]
    deg = degp_ref[0, :, :1] + degp_ref[1, :, :1]
    invd = 1.0 / jnp.maximum(deg, 1.0)
    meanx = (aggx_ref[0] + aggx_ref[1]) * invd
    z1 = jnp.maximum(
        jnp.dot(xt, we1r_ref[...], precision=hp)
        + jnp.dot(meanx, we1n_ref[...], precision=hp) + we1b_ref[...], 0.0)
    s1 = (jnp.dot(xt, wa1r_ref[...], precision=hp)
          + jnp.dot(meanx, wa1n_ref[...], precision=hp) + wa1b_ref[...])
    z1_ref[0] = z1[:, :D]
    z1_ref[1] = z1[:, D:]
    t16_ref[...] = jnp.concatenate(
        [jnp.dot(s1, wa2n_ref[...], precision=hp),
         jnp.zeros((TN, 16 - C), jnp.float32)], axis=1)
    r8_ref[...] = jnp.dot(s1, wa2r_ref[...], precision=hp)
    invd_ref[...] = jnp.broadcast_to(invd, (TN, 8))


def _tc1(x2, aggx, degp, We1_r, We1_n, We1_b, Wa1_r, Wa1_n, Wa1_b,
         Wa2_r, Wa2_n):
    full = lambda shape: pl.BlockSpec(shape, lambda i: (0,) * len(shape))
    return pl.pallas_call(
        _tc1_body,
        grid=(NT,),
        in_specs=[
            pl.BlockSpec((TN, D), lambda i: (i, 0)),
            pl.BlockSpec((NC, TN, D), lambda i: (0, i, 0)),
            pl.BlockSpec((NC, TN, 16), lambda i: (0, i, 0)),
            full((D, H)), full((D, H)), full((1, H)),
            full((D, H)), full((D, H)), full((1, H)),
            full((H, C)), full((H, C)),
        ],
        out_specs=[
            pl.BlockSpec((NC, TN, D), lambda i: (0, i, 0)),
            pl.BlockSpec((TN, 16), lambda i: (i, 0)),
            pl.BlockSpec((TN, C), lambda i: (i, 0)),
            pl.BlockSpec((TN, 8), lambda i: (i, 0)),
        ],
        out_shape=[
            jax.ShapeDtypeStruct((NC, N2, D), jnp.float32),
            jax.ShapeDtypeStruct((N2, 16), jnp.float32),
            jax.ShapeDtypeStruct((N2, C), jnp.float32),
            jax.ShapeDtypeStruct((N2, 8), jnp.float32),
        ],
    )(x2, aggx, degp, We1_r, We1_n, We1_b, Wa1_r, Wa1_n, Wa1_b, Wa2_r, Wa2_n)


def _tc2_body(z1_ref, aggz_ref, aggt_ref, r8_ref, invd_ref, batch_ref,
              we2r_ref, we2n_ref, we2b_ref, wa2b_ref,
              wc1_ref, bc1_ref, wc2_ref, bc2_ref,
              out_ref, pooled_ref):
    hp = jax.lax.Precision.HIGHEST
    i = pl.program_id(0)
    invd = invd_ref[:, :1]
    z1a = z1_ref[0]
    z1b = z1_ref[1]
    mza = aggz_ref[0] * invd
    mzb = aggz_ref[1] * invd
    z2 = jnp.maximum(
        jnp.dot(z1a, we2r_ref[:D, :], precision=hp)
        + jnp.dot(z1b, we2r_ref[D:, :], precision=hp)
        + jnp.dot(mza, we2n_ref[:D, :], precision=hp)
        + jnp.dot(mzb, we2n_ref[D:, :], precision=hp)
        + we2b_ref[...], 0.0)
    meant = (aggt_ref[0, :, :C] + aggt_ref[1, :, :C]) * invd
    s2 = r8_ref[...] + meant + wa2b_ref[...]
    sm = jnp.exp(s2 - jnp.max(s2, axis=-1, keepdims=True))
    sm = sm / jnp.sum(sm, axis=-1, keepdims=True)
    bt = batch_ref[0, 0, :]
    onehot = (bt[:, None] == lax.broadcasted_iota(jnp.int32, (TN, B), 1)
              ).astype(jnp.float32)

    @pl.when(i == 0)
    def _():
        pooled_ref[...] = jnp.zeros((C * B, H), jnp.float32)

    for cc in range(C):
        w = onehot * sm[:, cc:cc + 1]
        pooled_ref[pl.ds(cc * B, B), :] += lax.dot_general(
            w, z2, dimension_numbers=(((0,), (0,)), ((), ())), precision=hp)

    @pl.when(i == NT - 1)
    def _():
        acc = jnp.zeros((B, H), jnp.float32)
        for cc in range(C):
            acc += jnp.dot(pooled_ref[pl.ds(cc * B, B), :],
                           wc1_ref[pl.ds(cc * H, H), :], precision=hp)
        h = jnp.maximum(acc + bc1_ref[...], 0.0)
        out_ref[...] = (jnp.dot(h, wc2_ref[...], precision=hp)
                        + bc2_ref[...]).reshape(1, B)


def _tc2(z1, aggz, aggt, r8, invd, batch3,
         We2_r, We2_n, We2_b, Wa2_b, Wc1, bc1, Wc2, bc2):
    full = lambda shape: pl.BlockSpec(shape, lambda i: (0,) * len(shape))
    return pl.pallas_call(
        _tc2_body,
        grid=(NT,),
        in_specs=[
            pl.BlockSpec((NC, TN, D), lambda i: (0, i, 0)),
            pl.BlockSpec((NC, TN, D), lambda i: (0, i, 0)),
            pl.BlockSpec((NC, TN, 16), lambda i: (0, i, 0)),
            pl.BlockSpec((TN, C), lambda i: (i, 0)),
            pl.BlockSpec((TN, 8), lambda i: (i, 0)),
            pl.BlockSpec((1, 1, TN), lambda i: (i, 0, 0)),
            full((H, H)), full((H, H)), full((1, H)), full((1, C)),
            full((C * H, H)), full((1, H)), full((H, 1)), full((1, 1)),
        ],
        out_specs=pl.BlockSpec((1, B), lambda i: (0, 0)),
        out_shape=jax.ShapeDtypeStruct((1, B), jnp.float32),
        scratch_shapes=[pltpu.VMEM((C * B, H), jnp.float32)],
    )(z1, aggz, aggt, r8, invd, batch3,
      We2_r, We2_n, We2_b, Wa2_b, Wc1, bc1, Wc2, bc2)


def kernel(x, edge_index, batch, We1_r, We1_n, We1_b, We2_r, We2_n, We2_b,
           Wa1_r, Wa1_n, Wa1_b, Wa2_r, Wa2_n, Wa2_b, Wc1, bc1, Wc2, bc2):
    f32 = jnp.float32
    x2 = jnp.zeros((N2, D), f32).at[:N].set(x)
    pad = jnp.full((EPAD - E,), N2 - 1, jnp.int32)
    srcR = jnp.concatenate([edge_index[0].astype(jnp.int32), pad]
                           ).reshape(NROW, CH)
    dstR = jnp.concatenate([edge_index[1].astype(jnp.int32), pad]
                           ).reshape(NROW, CH)
    batch3 = jnp.full((N2,), B, jnp.int32).at[:N].set(
        batch.astype(jnp.int32)).reshape(NT, 1, TN)
    zrow = jnp.zeros((RPT, D), f32)
    zdeg = jnp.zeros((RPT, 16), f32)
    ones16 = jnp.ones((CH, 16), f32)

    degp = _sca(dstR, zdeg, ones16)
    aggx = _scb(x2, srcR, dstR, zrow)
    z1, t16, r8, invd = _tc1(x2, aggx, degp, We1_r, We1_n,
                             We1_b.reshape(1, H), Wa1_r, Wa1_n,
                             Wa1_b.reshape(1, H), Wa2_r, Wa2_n)
    aggz = _scc(z1.reshape(NC * N2, D), srcR, dstR, zrow)
    aggt = _scd(t16, srcR, dstR, zdeg)
    out = _tc2(z1, aggz, aggt, r8, invd, batch3,
               We2_r, We2_n, We2_b.reshape(1, H), Wa2_b.reshape(1, C),
               Wc1, bc1.reshape(1, H), Wc2, bc2.reshape(1, 1))
    return out[0]


# speedup vs baseline: 1.0200x; 1.0200x over previous
import functools

import jax
import jax.numpy as jnp
from jax import lax
from jax.experimental import pallas as pl
from jax.experimental.pallas import tpu as pltpu
from jax.experimental.pallas import tpu_sc as plsc

N = 10000
E = 320000
D = 128
H = 256
C = 8
B = 64

N2 = 10240
CH = 64
NB = 4
NG = 2
BLK = 16
NBT = 4
BLKT = 16
NC = 2
NS = 16
EPAD = 327680
NROW = EPAD // CH
RPT = N2 // NS
NCH1 = NROW // (NC * NS)
NCHZ = NROW // NS
TN = 512
NT = N2 // TN

@functools.lru_cache(maxsize=None)
def _mesh():
    return plsc.VectorSubcoreMesh(core_axis_name="c", subcore_axis_name="s",
                                  num_cores=NC, num_subcores=NS)


def _idx_block(src_hbm, dst_hbm, sed, ded, st, blkrow, blk, coff):
    pltpu.sync_copy(src_hbm.at[pl.ds(blkrow, blk)], sed.at[st])
    pltpu.sync_copy(dst_hbm.at[pl.ds(blkrow, blk)], ded.at[st])
    if coff is not None:
        for j in range(blk):
            for k in range(CH // 16):
                sl = pl.ds(k * 16, 16)
                sed[st, j, sl] = sed[st, j, sl] + coff


def _ring(table, src_hbm, dst_hbm, sed, ded, rows, gsems, ssems, acc,
          nb, ng, blk, base, nch, coff=None):
    nblk = nch // blk

    def ref(st, j, k):
        return (st, j - k) if j >= k else (1 - st, blk + j - k)

    def finalize(st1, j1, b):
        pltpu.make_async_copy(table.at[sed.at[st1, j1]], rows.at[b],
                              gsems[b]).wait()
        pltpu.async_copy(rows.at[b], acc.at[ded.at[st1, j1]], ssems[b],
                         add=True)

    def swait(st2, j2, b):
        pltpu.make_async_copy(rows.at[b], acc.at[ded.at[st2, j2]],
                              ssems[b]).wait()

    def block(st, blkrow, first):
        _idx_block(src_hbm, dst_hbm, sed, ded, st, blkrow, blk, coff)
        for j in range(blk):
            b = j % nb
            if not (first and j < ng):
                st1, j1 = ref(st, j, ng)
                finalize(st1, j1, (j - ng) % nb)
            if not (first and j < nb):
                st2, j2 = ref(st, j, nb)
                swait(st2, j2, b)
            pltpu.async_copy(table.at[sed.at[st, j]], rows.at[b], gsems[b])

    block(0, base, True)
    block(1, base + blk, False)

    def pair(p, carry):
        row0 = base + 2 * p * blk
        block(0, row0, False)
        block(1, row0 + blk, False)
        return carry

    lax.fori_loop(1, nblk // 2, pair, 0)
    for j in range(blk, blk + ng):
        finalize(1, j - ng, (j - ng) % nb)
    for j in range(blk - nb, blk):
        swait(1, j, j % nb)


def _sca_body(dst_hbm, zdeg_hbm, ones_hbm, degp_hbm, accd, ded, ones_v,
              sem0, sem1):
    c = lax.axis_index("c")
    s = lax.axis_index("s")
    sems = (sem0, sem1)
    pltpu.sync_copy(zdeg_hbm, accd.at[pl.ds(s * RPT, RPT)])
    pltpu.sync_copy(ones_hbm, ones_v)
    plsc.subcore_barrier()

    base = (c * NS + s) * NCH1
    nblk = NCH1 // BLK
    na = 2

    def swait(st, j, b):
        pltpu.make_async_copy(ones_v, accd.at[ded.at[st, j]], sems[b]).wait()

    def block(st, blkrow, first):
        pltpu.sync_copy(dst_hbm.at[pl.ds(blkrow, BLK)], ded.at[st])
        for j in range(BLK):
            b = j % na
            if first and j < na:
                pass
            elif j < na:
                swait(1 - st, BLK - na + j, b)
            else:
                swait(st, j - na, b)
            pltpu.async_copy(ones_v, accd.at[ded.at[st, j]], sems[b],
                             add=True)

    block(0, base, True)
    block(1, base + BLK, False)

    def pair(p, carry):
        row0 = base + 2 * p * BLK
        block(0, row0, False)
        block(1, row0 + BLK, False)
        return carry

    lax.fori_loop(1, nblk // 2, pair, 0)
    for j in range(BLK - na, BLK):
        swait(1, j, j % na)

    plsc.subcore_barrier()
    pltpu.sync_copy(accd.at[pl.ds(s * RPT, RPT)],
                    degp_hbm.at[c, pl.ds(s * RPT, RPT)])


@functools.lru_cache(maxsize=None)
def _make_sca():
    return pl.kernel(
        _sca_body,
        out_type=jax.ShapeDtypeStruct((NC, N2, 16), jnp.float32),
        mesh=_mesh(),
        compiler_params=pltpu.CompilerParams(use_tc_tiling_on_sc=False),
        scratch_types=[
            pltpu.VMEM_SHARED((N2, 16), jnp.float32),
            pltpu.VMEM((2, BLK, CH), jnp.int32),
            pltpu.VMEM((CH, 16), jnp.float32),
            pltpu.SemaphoreType.DMA,
            pltpu.SemaphoreType.DMA,
        ],
    )


def _sca(*args):
    return _make_sca()(*args)


def _scw_body(table_hbm, src_hbm, dst_hbm, zrow_hbm, agg_hbm,
              acc, sed, ded, rows, g0, g1, g2, g3, s0, s1, s2, s3,
              *, split32, use_coff):
    c = lax.axis_index("c")
    s = lax.axis_index("s")
    pltpu.sync_copy(zrow_hbm, acc.at[pl.ds(s * RPT, RPT)])
    plsc.subcore_barrier()

    if split32:
        base = (c * NS + s) * NCH1
        nch = NCH1
    else:
        base = s * NCHZ
        nch = NCHZ
    _ring(table_hbm, src_hbm, dst_hbm, sed, ded, rows, (g0, g1, g2, g3),
          (s0, s1, s2, s3), acc, NB, NG, BLK, base, nch,
          coff=(c * N2 if use_coff else None))

    plsc.subcore_barrier()
    pltpu.sync_copy(acc.at[pl.ds(s * RPT, RPT)],
                    agg_hbm.at[c, pl.ds(s * RPT, RPT)])


@functools.lru_cache(maxsize=None)
def _make_scw(split32, use_coff):
    body = functools.partial(_scw_body, split32=split32, use_coff=use_coff)
    return pl.kernel(
        body,
        out_type=jax.ShapeDtypeStruct((NC, N2, D), jnp.float32),
        mesh=_mesh(),
        compiler_params=pltpu.CompilerParams(use_tc_tiling_on_sc=False),
        scratch_types=[
            pltpu.VMEM_SHARED((N2, D), jnp.float32),
            pltpu.VMEM((2, BLK, CH), jnp.int32),
            pltpu.VMEM((2, BLK, CH), jnp.int32),
            pltpu.VMEM((NB, CH, D), jnp.float32),
        ] + [pltpu.SemaphoreType.DMA] * 8,
    )


def _scb(x2, srcR, dstR, zrow):
    return _make_scw(True, False)(x2, srcR, dstR, zrow)


def _scc(z1f, srcR, dstR, zrow):
    return _make_scw(False, True)(z1f, srcR, dstR, zrow)


def _scd_body(t16_hbm, src_hbm, dst_hbm, zdeg_hbm, aggt_hbm,
              acct, sed, ded, trows, g0, g1, g2, g3, s0, s1, s2, s3):
    c = lax.axis_index("c")
    s = lax.axis_index("s")
    pltpu.sync_copy(zdeg_hbm, acct.at[pl.ds(s * RPT, RPT)])
    plsc.subcore_barrier()

    _ring(t16_hbm, src_hbm, dst_hbm, sed, ded, trows, (g0, g1, g2, g3),
          (s0, s1, s2, s3), acct, NBT, 2, BLKT, (c * NS + s) * NCH1, NCH1)

    plsc.subcore_barrier()
    pltpu.sync_copy(acct.at[pl.ds(s * RPT, RPT)],
                    aggt_hbm.at[c, pl.ds(s * RPT, RPT)])


@functools.lru_cache(maxsize=None)
def _make_scd():
    return pl.kernel(
        _scd_body,
        out_type=jax.ShapeDtypeStruct((NC, N2, 16), jnp.float32),
        mesh=_mesh(),
        compiler_params=pltpu.CompilerParams(use_tc_tiling_on_sc=False),
        scratch_types=[
            pltpu.VMEM_SHARED((N2, 16), jnp.float32),
            pltpu.VMEM((2, BLKT, CH), jnp.int32),
            pltpu.VMEM((2, BLKT, CH), jnp.int32),
            pltpu.VMEM((NBT, CH, 16), jnp.float32),
        ] + [pltpu.SemaphoreType.DMA] * 8,
    )


def _scd(*args):
    return _make_scd()(*args)


def _tc1_body(x_ref, aggx_ref, degp_ref,
              we1r_ref, we1n_ref, we1b_ref, wa1r_ref, wa1n_ref, wa1b_ref,
              wa2r_ref, wa2n_ref,
              z1_ref, t16_ref, r8_ref, invd_ref):
    hp = jax.lax.Precision.HIGHEST
    xt = x_ref[...]
    deg = degp_ref[0, :, :1] + degp_ref[1, :, :1]
    invd = 1.0 / jnp.maximum(deg, 1.0)
    meanx = (aggx_ref[0] + aggx_ref[1]) * invd
    z1 = jnp.maximum(
        jnp.dot(xt, we1r_ref[...], precision=hp)
        + jnp.dot(meanx, we1n_ref[...], precision=hp) + we1b_ref[...], 0.0)
    s1 = (jnp.dot(xt, wa1r_ref[...], precision=hp)
          + jnp.dot(meanx, wa1n_ref[...], precision=hp) + wa1b_ref[...])
    z1_ref[0] = z1[:, :D]
    z1_ref[1] = z1[:, D:]
    t16_ref[...] = jnp.concatenate(
        [jnp.dot(s1, wa2n_ref[...], precision=hp),
         jnp.zeros((TN, 16 - C), jnp.float32)], axis=1)
    r8_ref[...] = jnp.dot(s1, wa2r_ref[...], precision=hp)
    invd_ref[...] = jnp.broadcast_to(invd, (TN, 8))


def _tc1(x2, aggx, degp, We1_r, We1_n, We1_b, Wa1_r, Wa1_n, Wa1_b,
         Wa2_r, Wa2_n):
    full = lambda shape: pl.BlockSpec(shape, lambda i: (0,) * len(shape))
    return pl.pallas_call(
        _tc1_body,
        grid=(NT,),
        in_specs=[
            pl.BlockSpec((TN, D), lambda i: (i, 0)),
            pl.BlockSpec((NC, TN, D), lambda i: (0, i, 0)),
            pl.BlockSpec((NC, TN, 16), lambda i: (0, i, 0)),
            full((D, H)), full((D, H)), full((1, H)),
            full((D, H)), full((D, H)), full((1, H)),
            full((H, C)), full((H, C)),
        ],
        out_specs=[
            pl.BlockSpec((NC, TN, D), lambda i: (0, i, 0)),
            pl.BlockSpec((TN, 16), lambda i: (i, 0)),
            pl.BlockSpec((TN, C), lambda i: (i, 0)),
            pl.BlockSpec((TN, 8), lambda i: (i, 0)),
        ],
        out_shape=[
            jax.ShapeDtypeStruct((NC, N2, D), jnp.float32),
            jax.ShapeDtypeStruct((N2, 16), jnp.float32),
            jax.ShapeDtypeStruct((N2, C), jnp.float32),
            jax.ShapeDtypeStruct((N2, 8), jnp.float32),
        ],
    )(x2, aggx, degp, We1_r, We1_n, We1_b, Wa1_r, Wa1_n, Wa1_b, Wa2_r, Wa2_n)


def _tc2_body(z1_ref, aggz_ref, aggt_ref, r8_ref, invd_ref, batch_ref,
              we2r_ref, we2n_ref, we2b_ref, wa2b_ref,
              wc1_ref, bc1_ref, wc2_ref, bc2_ref,
              out_ref, pooled_ref):
    hp = jax.lax.Precision.HIGHEST
    i = pl.program_id(0)
    invd = invd_ref[:, :1]
    z1a = z1_ref[0]
    z1b = z1_ref[1]
    mza = aggz_ref[0] * invd
    mzb = aggz_ref[1] * invd
    z2 = jnp.maximum(
        jnp.dot(z1a, we2r_ref[:D, :], precision=hp)
        + jnp.dot(z1b, we2r_ref[D:, :], precision=hp)
        + jnp.dot(mza, we2n_ref[:D, :], precision=hp)
        + jnp.dot(mzb, we2n_ref[D:, :], precision=hp)
        + we2b_ref[...], 0.0)
    meant = (aggt_ref[0, :, :C] + aggt_ref[1, :, :C]) * invd
    s2 = r8_ref[...] + meant + wa2b_ref[...]
    sm = jnp.exp(s2 - jnp.max(s2, axis=-1, keepdims=True))
    sm = sm / jnp.sum(sm, axis=-1, keepdims=True)
    bt = batch_ref[0, 0, :]
    onehot = (bt[:, None] == lax.broadcasted_iota(jnp.int32, (TN, B), 1)
              ).astype(jnp.float32)

    @pl.when(i == 0)
    def _():
        pooled_ref[...] = jnp.zeros((C * B, H), jnp.float32)

    for cc in range(C):
        w = onehot * sm[:, cc:cc + 1]
        pooled_ref[pl.ds(cc * B, B), :] += lax.dot_general(
            w, z2, dimension_numbers=(((0,), (0,)), ((), ())), precision=hp)

    @pl.when(i == NT - 1)
    def _():
        acc = jnp.zeros((B, H), jnp.float32)
        for cc in range(C):
            acc += jnp.dot(pooled_ref[pl.ds(cc * B, B), :],
                           wc1_ref[pl.ds(cc * H, H), :], precision=hp)
        h = jnp.maximum(acc + bc1_ref[...], 0.0)
        out_ref[...] = (jnp.dot(h, wc2_ref[...], precision=hp)
                        + bc2_ref[...]).reshape(1, B)


def _tc2(z1, aggz, aggt, r8, invd, batch3,
         We2_r, We2_n, We2_b, Wa2_b, Wc1, bc1, Wc2, bc2):
    full = lambda shape: pl.BlockSpec(shape, lambda i: (0,) * len(shape))
    return pl.pallas_call(
        _tc2_body,
        grid=(NT,),
        in_specs=[
            pl.BlockSpec((NC, TN, D), lambda i: (0, i, 0)),
            pl.BlockSpec((NC, TN, D), lambda i: (0, i, 0)),
            pl.BlockSpec((NC, TN, 16), lambda i: (0, i, 0)),
            pl.BlockSpec((TN, C), lambda i: (i, 0)),
            pl.BlockSpec((TN, 8), lambda i: (i, 0)),
            pl.BlockSpec((1, 1, TN), lambda i: (i, 0, 0)),
            full((H, H)), full((H, H)), full((1, H)), full((1, C)),
            full((C * H, H)), full((1, H)), full((H, 1)), full((1, 1)),
        ],
        out_specs=pl.BlockSpec((1, B), lambda i: (0, 0)),
        out_shape=jax.ShapeDtypeStruct((1, B), jnp.float32),
        scratch_shapes=[pltpu.VMEM((C * B, H), jnp.float32)],
    )(z1, aggz, aggt, r8, invd, batch3,
      We2_r, We2_n, We2_b, Wa2_b, Wc1, bc1, Wc2, bc2)


def kernel(x, edge_index, batch, We1_r, We1_n, We1_b, We2_r, We2_n, We2_b,
           Wa1_r, Wa1_n, Wa1_b, Wa2_r, Wa2_n, Wa2_b, Wc1, bc1, Wc2, bc2):
    f32 = jnp.float32
    x2 = jnp.zeros((N2, D), f32).at[:N].set(x)
    pad = jnp.full((EPAD - E,), N2 - 1, jnp.int32)
    srcR = jnp.concatenate([edge_index[0].astype(jnp.int32), pad]
                           ).reshape(NROW, CH)
    dstR = jnp.concatenate([edge_index[1].astype(jnp.int32), pad]
                           ).reshape(NROW, CH)
    batch3 = jnp.full((N2,), B, jnp.int32).at[:N].set(
        batch.astype(jnp.int32)).reshape(NT, 1, TN)
    zrow = jnp.zeros((RPT, D), f32)
    zdeg = jnp.zeros((RPT, 16), f32)
    ones16 = jnp.ones((CH, 16), f32)

    degp = _sca(dstR, zdeg, ones16)
    aggx = _scb(x2, srcR, dstR, zrow)
    z1, t16, r8, invd = _tc1(x2, aggx, degp, We1_r, We1_n,
                             We1_b.reshape(1, H), Wa1_r, Wa1_n,
                             Wa1_b.reshape(1, H), Wa2_r, Wa2_n)
    aggz = _scc(z1.reshape(NC * N2, D), srcR, dstR, zrow)
    aggt = _scd(t16, srcR, dstR, zdeg)
    out = _tc2(z1, aggz, aggt, r8, invd, batch3,
               We2_r, We2_n, We2_b.reshape(1, H), Wa2_b.reshape(1, C),
               Wc1, bc1.reshape(1, H), Wc2, bc2.reshape(1, 1))
    return out[0]


# speedup vs baseline: 1.9391x; 1.9011x over previous
import functools

import jax
import jax.numpy as jnp
from jax import lax
from jax.experimental import pallas as pl
from jax.experimental.pallas import tpu as pltpu
from jax.experimental.pallas import tpu_sc as plsc

N = 10000
E = 320000
D = 128
H = 256
C = 8
B = 64

N2 = 10240
CH = 64
NB = 4
NG = 2
BLK = 16
NBT = 4
BLKT = 16
NC = 2
NS = 16
EPAD = 327680
NROW = EPAD // CH
RPT = N2 // NS
NCH1 = NROW // (NC * NS)
NCHZ = NROW // NS
TN = 512
NT = N2 // TN

@functools.lru_cache(maxsize=None)
def _mesh():
    return plsc.VectorSubcoreMesh(core_axis_name="c", subcore_axis_name="s",
                                  num_cores=NC, num_subcores=NS)


def _idx_block(src_hbm, dst_hbm, sed, ded, st, blkrow, blk, coff):
    pltpu.sync_copy(src_hbm.at[pl.ds(blkrow, blk)], sed.at[st])
    pltpu.sync_copy(dst_hbm.at[pl.ds(blkrow, blk)], ded.at[st])
    if coff is not None:
        for j in range(blk):
            for k in range(CH // 16):
                sl = pl.ds(k * 16, 16)
                sed[st, j, sl] = sed[st, j, sl] + coff


def _ring(table, src_hbm, dst_hbm, sed, ded, rows, gsems, ssems, acc,
          nb, ng, blk, base, nch, coff=None):
    nblk = nch // blk

    def ref(st, j, k):
        return (st, j - k) if j >= k else (1 - st, blk + j - k)

    def finalize(st1, j1, b):
        pltpu.make_async_copy(table.at[sed.at[st1, j1]], rows.at[b],
                              gsems[b]).wait()
        pltpu.async_copy(rows.at[b], acc.at[ded.at[st1, j1]], ssems[b],
                         add=True)

    def swait(st2, j2, b):
        pltpu.make_async_copy(rows.at[b], acc.at[ded.at[st2, j2]],
                              ssems[b]).wait()

    def block(st, blkrow, first):
        _idx_block(src_hbm, dst_hbm, sed, ded, st, blkrow, blk, coff)
        for j in range(blk):
            b = j % nb
            if not (first and j < ng):
                st1, j1 = ref(st, j, ng)
                finalize(st1, j1, (j - ng) % nb)
            if not (first and j < nb):
                st2, j2 = ref(st, j, nb)
                swait(st2, j2, b)
            pltpu.async_copy(table.at[sed.at[st, j]], rows.at[b], gsems[b])

    block(0, base, True)
    block(1, base + blk, False)

    def pair(p, carry):
        row0 = base + 2 * p * blk
        block(0, row0, False)
        block(1, row0 + blk, False)
        return carry

    lax.fori_loop(1, nblk // 2, pair, 0)
    for j in range(blk, blk + ng):
        finalize(1, j - ng, (j - ng) % nb)
    for j in range(blk - nb, blk):
        swait(1, j, j % nb)


def _sca_body(dst_hbm, zdeg_hbm, ones_hbm, degp_hbm, accd, ded, ones_v,
              sem0, sem1):
    c = lax.axis_index("c")
    s = lax.axis_index("s")
    sems = (sem0, sem1)
    pltpu.sync_copy(zdeg_hbm, accd.at[pl.ds(s * RPT, RPT)])
    pltpu.sync_copy(ones_hbm, ones_v)
    plsc.subcore_barrier()

    base = (c * NS + s) * NCH1
    nblk = NCH1 // BLK
    na = 2

    def swait(st, j, b):
        pltpu.make_async_copy(ones_v, accd.at[ded.at[st, j]], sems[b]).wait()

    def block(st, blkrow, first):
        pltpu.sync_copy(dst_hbm.at[pl.ds(blkrow, BLK)], ded.at[st])
        for j in range(BLK):
            b = j % na
            if first and j < na:
                pass
            elif j < na:
                swait(1 - st, BLK - na + j, b)
            else:
                swait(st, j - na, b)
            pltpu.async_copy(ones_v, accd.at[ded.at[st, j]], sems[b],
                             add=True)

    block(0, base, True)
    block(1, base + BLK, False)

    def pair(p, carry):
        row0 = base + 2 * p * BLK
        block(0, row0, False)
        block(1, row0 + BLK, False)
        return carry

    lax.fori_loop(1, nblk // 2, pair, 0)
    for j in range(BLK - na, BLK):
        swait(1, j, j % na)

    plsc.subcore_barrier()
    pltpu.sync_copy(accd.at[pl.ds(s * RPT, RPT)],
                    degp_hbm.at[c, pl.ds(s * RPT, RPT)])


@functools.lru_cache(maxsize=None)
def _make_sca():
    return pl.kernel(
        _sca_body,
        out_type=jax.ShapeDtypeStruct((NC, N2, 16), jnp.float32),
        mesh=_mesh(),
        compiler_params=pltpu.CompilerParams(use_tc_tiling_on_sc=False),
        scratch_types=[
            pltpu.VMEM_SHARED((N2, 16), jnp.float32),
            pltpu.VMEM((2, BLK, CH), jnp.int32),
            pltpu.VMEM((CH, 16), jnp.float32),
            pltpu.SemaphoreType.DMA,
            pltpu.SemaphoreType.DMA,
        ],
    )


def _sca(*args):
    return _make_sca()(*args)


def _scw_body(table_hbm, src_hbm, dst_hbm, zrow_hbm, agg_hbm,
              acc, sed, ded, rows, g0, g1, g2, g3, s0, s1, s2, s3,
              *, split32, use_coff):
    c = lax.axis_index("c")
    s = lax.axis_index("s")
    pltpu.sync_copy(zrow_hbm, acc.at[pl.ds(s * RPT, RPT)])
    plsc.subcore_barrier()

    if split32:
        base = (c * NS + s) * NCH1
        nch = NCH1
    else:
        base = s * NCHZ
        nch = NCHZ
    _ring(table_hbm, src_hbm, dst_hbm, sed, ded, rows, (g0, g1, g2, g3),
          (s0, s1, s2, s3), acc, NB, NG, BLK, base, nch,
          coff=(c * N2 if use_coff else None))

    plsc.subcore_barrier()
    pltpu.sync_copy(acc.at[pl.ds(s * RPT, RPT)],
                    agg_hbm.at[c, pl.ds(s * RPT, RPT)])


@functools.lru_cache(maxsize=None)
def _make_scw(split32, use_coff):
    body = functools.partial(_scw_body, split32=split32, use_coff=use_coff)
    return pl.kernel(
        body,
        out_type=jax.ShapeDtypeStruct((NC, N2, D), jnp.float32),
        mesh=_mesh(),
        compiler_params=pltpu.CompilerParams(use_tc_tiling_on_sc=False),
        scratch_types=[
            pltpu.VMEM_SHARED((N2, D), jnp.float32),
            pltpu.VMEM((2, BLK, CH), jnp.int32),
            pltpu.VMEM((2, BLK, CH), jnp.int32),
            pltpu.VMEM((NB, CH, D), jnp.float32),
        ] + [pltpu.SemaphoreType.DMA] * 8,
    )


def _scb(x2, srcR, dstR, zrow):
    return _make_scw(True, False)(x2, srcR, dstR, zrow)


def _scc(z1f, srcR, dstR, zrow):
    return _make_scw(False, True)(z1f, srcR, dstR, zrow)


def _scd_body(t16_hbm, src_hbm, dst_hbm, zdeg_hbm, aggt_hbm,
              acct, sed, ded, trows, g0, g1, g2, g3, s0, s1, s2, s3):
    c = lax.axis_index("c")
    s = lax.axis_index("s")
    pltpu.sync_copy(zdeg_hbm, acct.at[pl.ds(s * RPT, RPT)])
    plsc.subcore_barrier()

    _ring(t16_hbm, src_hbm, dst_hbm, sed, ded, trows, (g0, g1, g2, g3),
          (s0, s1, s2, s3), acct, NBT, 2, BLKT, (c * NS + s) * NCH1, NCH1)

    plsc.subcore_barrier()
    pltpu.sync_copy(acct.at[pl.ds(s * RPT, RPT)],
                    aggt_hbm.at[c, pl.ds(s * RPT, RPT)])


@functools.lru_cache(maxsize=None)
def _make_scd():
    return pl.kernel(
        _scd_body,
        out_type=jax.ShapeDtypeStruct((NC, N2, 16), jnp.float32),
        mesh=_mesh(),
        compiler_params=pltpu.CompilerParams(use_tc_tiling_on_sc=False),
        scratch_types=[
            pltpu.VMEM_SHARED((N2, 16), jnp.float32),
            pltpu.VMEM((2, BLKT, CH), jnp.int32),
            pltpu.VMEM((2, BLKT, CH), jnp.int32),
            pltpu.VMEM((NBT, CH, 16), jnp.float32),
        ] + [pltpu.SemaphoreType.DMA] * 8,
    )


def _scd(*args):
    return _make_scd()(*args)


def _tc1_body(x_ref, aggx_ref, degp_ref,
              we1r_ref, we1n_ref, we1b_ref, wa1r_ref, wa1n_ref, wa1b_ref,
              wa2r_ref, wa2n_ref,
              z1_ref, t16_ref, r8_ref, invd_ref):
    hp = jax.lax.Precision.HIGHEST
    xt = x_ref[...]
    deg = degp_ref[0, :, :1] + degp_ref[1, :, :1]
    invd = 1.0 / jnp.maximum(deg, 1.0)
    meanx = (aggx_ref[0] + aggx_ref[1]) * invd
    z1 = jnp.maximum(
        jnp.dot(xt, we1r_ref[...], precision=hp)
        + jnp.dot(meanx, we1n_ref[...], precision=hp) + we1b_ref[...], 0.0)
    s1 = (jnp.dot(xt, wa1r_ref[...], precision=hp)
          + jnp.dot(meanx, wa1n_ref[...], precision=hp) + wa1b_ref[...])
    z1_ref[0] = z1[:, :D]
    z1_ref[1] = z1[:, D:]
    t16_ref[...] = jnp.concatenate(
        [jnp.dot(s1, wa2n_ref[...], precision=hp),
         jnp.zeros((TN, 16 - C), jnp.float32)], axis=1)
    r8_ref[...] = jnp.dot(s1, wa2r_ref[...], precision=hp)
    invd_ref[...] = jnp.broadcast_to(invd, (TN, 8))


def _tc1(x2, aggx, degp, We1_r, We1_n, We1_b, Wa1_r, Wa1_n, Wa1_b,
         Wa2_r, Wa2_n):
    full = lambda shape: pl.BlockSpec(shape, lambda i: (0,) * len(shape))
    return pl.pallas_call(
        _tc1_body,
        grid=(NT,),
        in_specs=[
            pl.BlockSpec((TN, D), lambda i: (i, 0)),
            pl.BlockSpec((NC, TN, D), lambda i: (0, i, 0)),
            pl.BlockSpec((NC, TN, 16), lambda i: (0, i, 0)),
            full((D, H)), full((D, H)), full((1, H)),
            full((D, H)), full((D, H)), full((1, H)),
            full((H, C)), full((H, C)),
        ],
        out_specs=[
            pl.BlockSpec((NC, TN, D), lambda i: (0, i, 0)),
            pl.BlockSpec((TN, 16), lambda i: (i, 0)),
            pl.BlockSpec((TN, C), lambda i: (i, 0)),
            pl.BlockSpec((TN, 8), lambda i: (i, 0)),
        ],
        out_shape=[
            jax.ShapeDtypeStruct((NC, N2, D), jnp.float32),
            jax.ShapeDtypeStruct((N2, 16), jnp.float32),
            jax.ShapeDtypeStruct((N2, C), jnp.float32),
            jax.ShapeDtypeStruct((N2, 8), jnp.float32),
        ],
    )(x2, aggx, degp, We1_r, We1_n, We1_b, Wa1_r, Wa1_n, Wa1_b, Wa2_r, Wa2_n)


def _tc2_body(z1_ref, aggz_ref, aggt_ref, r8_ref, invd_ref, batch_ref,
              we2r_ref, we2n_ref, we2b_ref, wa2b_ref,
              wc1_ref, bc1_ref, wc2_ref, bc2_ref,
              out_ref, pooled_ref):
    hp = jax.lax.Precision.HIGHEST
    i = pl.program_id(0)
    invd = invd_ref[:, :1]
    z1a = z1_ref[0]
    z1b = z1_ref[1]
    mza = aggz_ref[0] * invd
    mzb = aggz_ref[1] * invd
    z2 = jnp.maximum(
        jnp.dot(z1a, we2r_ref[:D, :], precision=hp)
        + jnp.dot(z1b, we2r_ref[D:, :], precision=hp)
        + jnp.dot(mza, we2n_ref[:D, :], precision=hp)
        + jnp.dot(mzb, we2n_ref[D:, :], precision=hp)
        + we2b_ref[...], 0.0)
    meant = (aggt_ref[0, :, :C] + aggt_ref[1, :, :C]) * invd
    s2 = r8_ref[...] + meant + wa2b_ref[...]
    sm = jnp.exp(s2 - jnp.max(s2, axis=-1, keepdims=True))
    sm = sm / jnp.sum(sm, axis=-1, keepdims=True)
    bt = batch_ref[0, 0, :]
    onehot = (bt[:, None] == lax.broadcasted_iota(jnp.int32, (TN, B), 1)
              ).astype(jnp.float32)

    @pl.when(i == 0)
    def _():
        pooled_ref[...] = jnp.zeros((C * B, H), jnp.float32)

    for cc in range(C):
        w = onehot * sm[:, cc:cc + 1]
        pooled_ref[pl.ds(cc * B, B), :] += lax.dot_general(
            w, z2, dimension_numbers=(((0,), (0,)), ((), ())), precision=hp)

    @pl.when(i == NT - 1)
    def _():
        acc = jnp.zeros((B, H), jnp.float32)
        for cc in range(C):
            acc += jnp.dot(pooled_ref[pl.ds(cc * B, B), :],
                           wc1_ref[pl.ds(cc * H, H), :], precision=hp)
        h = jnp.maximum(acc + bc1_ref[...], 0.0)
        out_ref[...] = (jnp.dot(h, wc2_ref[...], precision=hp)
                        + bc2_ref[...]).reshape(1, B)


def _tc2(z1, aggz, aggt, r8, invd, batch3,
         We2_r, We2_n, We2_b, Wa2_b, Wc1, bc1, Wc2, bc2):
    full = lambda shape: pl.BlockSpec(shape, lambda i: (0,) * len(shape))
    return pl.pallas_call(
        _tc2_body,
        grid=(NT,),
        in_specs=[
            pl.BlockSpec((NC, TN, D), lambda i: (0, i, 0)),
            pl.BlockSpec((NC, TN, D), lambda i: (0, i, 0)),
            pl.BlockSpec((NC, TN, 16), lambda i: (0, i, 0)),
            pl.BlockSpec((TN, C), lambda i: (i, 0)),
            pl.BlockSpec((TN, 8), lambda i: (i, 0)),
            pl.BlockSpec((1, 1, TN), lambda i: (i, 0, 0)),
            full((H, H)), full((H, H)), full((1, H)), full((1, C)),
            full((C * H, H)), full((1, H)), full((H, 1)), full((1, 1)),
        ],
        out_specs=pl.BlockSpec((1, B), lambda i: (0, 0)),
        out_shape=jax.ShapeDtypeStruct((1, B), jnp.float32),
        scratch_shapes=[pltpu.VMEM((C * B, H), jnp.float32)],
    )(z1, aggz, aggt, r8, invd, batch3,
      We2_r, We2_n, We2_b, Wa2_b, Wc1, bc1, Wc2, bc2)


def kernel(x, edge_index, batch, We1_r, We1_n, We1_b, We2_r, We2_n, We2_b,
           Wa1_r, Wa1_n, Wa1_b, Wa2_r, Wa2_n, Wa2_b, Wc1, bc1, Wc2, bc2):
    f32 = jnp.float32
    x2 = jnp.zeros((N2, D), f32).at[:N].set(x)
    pad = N + jax.lax.iota(jnp.int32, EPAD - E) % (N2 - N)
    srcR = jnp.concatenate([edge_index[0].astype(jnp.int32), pad]
                           ).reshape(NROW, CH)
    dstR = jnp.concatenate([edge_index[1].astype(jnp.int32), pad]
                           ).reshape(NROW, CH)
    batch3 = jnp.full((N2,), B, jnp.int32).at[:N].set(
        batch.astype(jnp.int32)).reshape(NT, 1, TN)
    zrow = jnp.zeros((RPT, D), f32)
    zdeg = jnp.zeros((RPT, 16), f32)
    ones16 = jnp.ones((CH, 16), f32)

    degp = _sca(dstR, zdeg, ones16)
    aggx = _scb(x2, srcR, dstR, zrow)
    z1, t16, r8, invd = _tc1(x2, aggx, degp, We1_r, We1_n,
                             We1_b.reshape(1, H), Wa1_r, Wa1_n,
                             Wa1_b.reshape(1, H), Wa2_r, Wa2_n)
    aggz = _scc(z1.reshape(NC * N2, D), srcR, dstR, zrow)
    aggt = _scd(t16, srcR, dstR, zdeg)
    out = _tc2(z1, aggz, aggt, r8, invd, batch3,
               We2_r, We2_n, We2_b.reshape(1, H), Wa2_b.reshape(1, C),
               Wc1, bc1.reshape(1, H), Wc2, bc2.reshape(1, 1))
    return out[0]


# speedup vs baseline: 2.5927x; 1.3370x over previous
import functools

import jax
import jax.numpy as jnp
from jax import lax
from jax.experimental import pallas as pl
from jax.experimental.pallas import tpu as pltpu
from jax.experimental.pallas import tpu_sc as plsc

N = 10000
E = 320000
D = 128
H = 256
C = 8
B = 64

N2 = 10240
CH = 64
NB = 4
NG = 3
BLK = 16
NBT = 4
BLKT = 16
NC = 2
NS = 16
EPAD = 327680
NROW = EPAD // CH
RPT = N2 // NS
NCH1 = NROW // (NC * NS)
NCHZ = NROW // NS
TN = 512
NT = N2 // TN

@functools.lru_cache(maxsize=None)
def _mesh():
    return plsc.VectorSubcoreMesh(core_axis_name="c", subcore_axis_name="s",
                                  num_cores=NC, num_subcores=NS)


def _idx_block(src_hbm, dst_hbm, sed, ded, st, blkrow, blk, coff):
    pltpu.sync_copy(src_hbm.at[pl.ds(blkrow, blk)], sed.at[st])
    pltpu.sync_copy(dst_hbm.at[pl.ds(blkrow, blk)], ded.at[st])
    if coff is not None:
        for j in range(blk):
            for k in range(CH // 16):
                sl = pl.ds(k * 16, 16)
                sed[st, j, sl] = sed[st, j, sl] + coff


def _ring(table, src_hbm, dst_hbm, sed, ded, rows, gsems, ssems, acc,
          nb, ng, blk, base, nch, coff=None):
    nblk = nch // blk

    def ref(st, j, k):
        return (st, j - k) if j >= k else (1 - st, blk + j - k)

    def finalize(st1, j1, b):
        pltpu.make_async_copy(table.at[sed.at[st1, j1]], rows.at[b],
                              gsems[b]).wait()
        pltpu.async_copy(rows.at[b], acc.at[ded.at[st1, j1]], ssems[b],
                         add=True)

    def swait(st2, j2, b):
        pltpu.make_async_copy(rows.at[b], acc.at[ded.at[st2, j2]],
                              ssems[b]).wait()

    def block(st, blkrow, first):
        _idx_block(src_hbm, dst_hbm, sed, ded, st, blkrow, blk, coff)
        for j in range(blk):
            b = j % nb
            if not (first and j < ng):
                st1, j1 = ref(st, j, ng)
                finalize(st1, j1, (j - ng) % nb)
            if not (first and j < nb):
                st2, j2 = ref(st, j, nb)
                swait(st2, j2, b)
            pltpu.async_copy(table.at[sed.at[st, j]], rows.at[b], gsems[b])

    block(0, base, True)
    block(1, base + blk, False)

    def pair(p, carry):
        row0 = base + 2 * p * blk
        block(0, row0, False)
        block(1, row0 + blk, False)
        return carry

    lax.fori_loop(1, nblk // 2, pair, 0)
    for j in range(blk, blk + ng):
        finalize(1, j - ng, (j - ng) % nb)
    for j in range(blk - nb, blk):
        swait(1, j, j % nb)


def _sca_body(dst_hbm, zdeg_hbm, ones_hbm, degp_hbm, accd0, accd1, ded,
              ones_v, sem0, sem1):
    c = lax.axis_index("c")
    s = lax.axis_index("s")
    sems = (sem0, sem1)
    accs = (accd0, accd1)
    pltpu.sync_copy(zdeg_hbm, accd0.at[pl.ds(s * RPT, RPT)])
    pltpu.sync_copy(zdeg_hbm, accd1.at[pl.ds(s * RPT, RPT)])
    pltpu.sync_copy(ones_hbm, ones_v)
    plsc.subcore_barrier()

    base = (c * NS + s) * NCH1
    nblk = NCH1 // BLK
    na = 2

    def swait(st, j, b):
        pltpu.make_async_copy(ones_v, accs[b].at[ded.at[st, j]],
                              sems[b]).wait()

    def block(st, blkrow, first):
        pltpu.sync_copy(dst_hbm.at[pl.ds(blkrow, BLK)], ded.at[st])
        for j in range(BLK):
            b = j % na
            if first and j < na:
                pass
            elif j < na:
                swait(1 - st, BLK - na + j, b)
            else:
                swait(st, j - na, b)
            pltpu.async_copy(ones_v, accs[b].at[ded.at[st, j]], sems[b],
                             add=True)

    block(0, base, True)
    block(1, base + BLK, False)

    def pair(p, carry):
        row0 = base + 2 * p * BLK
        block(0, row0, False)
        block(1, row0 + BLK, False)
        return carry

    lax.fori_loop(1, nblk // 2, pair, 0)
    for j in range(BLK - na, BLK):
        swait(1, j, j % na)

    plsc.subcore_barrier()
    pltpu.sync_copy(accd0.at[pl.ds(s * RPT, RPT)],
                    degp_hbm.at[c, 0, pl.ds(s * RPT, RPT)])
    pltpu.sync_copy(accd1.at[pl.ds(s * RPT, RPT)],
                    degp_hbm.at[c, 1, pl.ds(s * RPT, RPT)])


@functools.lru_cache(maxsize=None)
def _make_sca():
    return pl.kernel(
        _sca_body,
        out_type=jax.ShapeDtypeStruct((NC, 2, N2, 16), jnp.float32),
        mesh=_mesh(),
        compiler_params=pltpu.CompilerParams(use_tc_tiling_on_sc=False),
        scratch_types=[
            pltpu.VMEM_SHARED((N2, 16), jnp.float32),
            pltpu.VMEM_SHARED((N2, 16), jnp.float32),
            pltpu.VMEM((2, BLK, CH), jnp.int32),
            pltpu.VMEM((CH, 16), jnp.float32),
            pltpu.SemaphoreType.DMA,
            pltpu.SemaphoreType.DMA,
        ],
    )


def _sca(*args):
    return _make_sca()(*args)


def _scw_body(table_hbm, src_hbm, dst_hbm, zrow_hbm, agg_hbm,
              acc, sed, ded, rows, g0, g1, g2, g3, s0, s1, s2, s3,
              *, split32, use_coff):
    c = lax.axis_index("c")
    s = lax.axis_index("s")
    pltpu.sync_copy(zrow_hbm, acc.at[pl.ds(s * RPT, RPT)])
    plsc.subcore_barrier()

    if split32:
        base = (c * NS + s) * NCH1
        nch = NCH1
    else:
        base = s * NCHZ
        nch = NCHZ
    _ring(table_hbm, src_hbm, dst_hbm, sed, ded, rows, (g0, g1, g2, g3),
          (s0, s1, s2, s3), acc, NB, NG, BLK, base, nch,
          coff=(c * N2 if use_coff else None))

    plsc.subcore_barrier()
    pltpu.sync_copy(acc.at[pl.ds(s * RPT, RPT)],
                    agg_hbm.at[c, pl.ds(s * RPT, RPT)])


@functools.lru_cache(maxsize=None)
def _make_scw(split32, use_coff):
    body = functools.partial(_scw_body, split32=split32, use_coff=use_coff)
    return pl.kernel(
        body,
        out_type=jax.ShapeDtypeStruct((NC, N2, D), jnp.float32),
        mesh=_mesh(),
        compiler_params=pltpu.CompilerParams(use_tc_tiling_on_sc=False),
        scratch_types=[
            pltpu.VMEM_SHARED((N2, D), jnp.float32),
            pltpu.VMEM((2, BLK, CH), jnp.int32),
            pltpu.VMEM((2, BLK, CH), jnp.int32),
            pltpu.VMEM((NB, CH, D), jnp.float32),
        ] + [pltpu.SemaphoreType.DMA] * 8,
    )


def _scb(x2, srcR, dstR, zrow):
    return _make_scw(True, False)(x2, srcR, dstR, zrow)


def _scc(z1f, srcR, dstR, zrow):
    return _make_scw(False, True)(z1f, srcR, dstR, zrow)


def _scd_body(t16_hbm, src_hbm, dst_hbm, zdeg_hbm, aggt_hbm,
              acct, sed, ded, trows, g0, g1, g2, g3, s0, s1, s2, s3):
    c = lax.axis_index("c")
    s = lax.axis_index("s")
    pltpu.sync_copy(zdeg_hbm, acct.at[pl.ds(s * RPT, RPT)])
    plsc.subcore_barrier()

    _ring(t16_hbm, src_hbm, dst_hbm, sed, ded, trows, (g0, g1, g2, g3),
          (s0, s1, s2, s3), acct, NBT, NBT - 1, BLKT, (c * NS + s) * NCH1,
          NCH1)

    plsc.subcore_barrier()
    pltpu.sync_copy(acct.at[pl.ds(s * RPT, RPT)],
                    aggt_hbm.at[c, pl.ds(s * RPT, RPT)])


@functools.lru_cache(maxsize=None)
def _make_scd():
    return pl.kernel(
        _scd_body,
        out_type=jax.ShapeDtypeStruct((NC, N2, 16), jnp.float32),
        mesh=_mesh(),
        compiler_params=pltpu.CompilerParams(use_tc_tiling_on_sc=False),
        scratch_types=[
            pltpu.VMEM_SHARED((N2, 16), jnp.float32),
            pltpu.VMEM((2, BLKT, CH), jnp.int32),
            pltpu.VMEM((2, BLKT, CH), jnp.int32),
            pltpu.VMEM((NBT, CH, 16), jnp.float32),
        ] + [pltpu.SemaphoreType.DMA] * 8,
    )


def _scd(*args):
    return _make_scd()(*args)


def _tc1_body(x_ref, aggx_ref, degp_ref,
              we1r_ref, we1n_ref, we1b_ref, wa1r_ref, wa1n_ref, wa1b_ref,
              wa2r_ref, wa2n_ref,
              z1_ref, t16_ref, r8_ref, invd_ref):
    xt = x_ref[...]
    deg = (degp_ref[0, :, :1] + degp_ref[1, :, :1]
           + degp_ref[2, :, :1] + degp_ref[3, :, :1])
    invd = 1.0 / jnp.maximum(deg, 1.0)
    meanx = (aggx_ref[0] + aggx_ref[1]) * invd
    z1 = jnp.maximum(
        jnp.dot(xt, we1r_ref[...])
        + jnp.dot(meanx, we1n_ref[...]) + we1b_ref[...], 0.0)
    s1 = (jnp.dot(xt, wa1r_ref[...])
          + jnp.dot(meanx, wa1n_ref[...]) + wa1b_ref[...])
    z1_ref[0] = z1[:, :D]
    z1_ref[1] = z1[:, D:]
    t16_ref[...] = jnp.concatenate(
        [jnp.dot(s1, wa2n_ref[...]),
         jnp.zeros((TN, 16 - C), jnp.float32)], axis=1)
    r8_ref[...] = jnp.dot(s1, wa2r_ref[...])
    invd_ref[...] = jnp.broadcast_to(invd, (TN, 8))


def _tc1(x2, aggx, degp, We1_r, We1_n, We1_b, Wa1_r, Wa1_n, Wa1_b,
         Wa2_r, Wa2_n):
    full = lambda shape: pl.BlockSpec(shape, lambda i: (0,) * len(shape))
    return pl.pallas_call(
        _tc1_body,
        grid=(NT,),
        in_specs=[
            pl.BlockSpec((TN, D), lambda i: (i, 0)),
            pl.BlockSpec((NC, TN, D), lambda i: (0, i, 0)),
            pl.BlockSpec((2 * NC, TN, 16), lambda i: (0, i, 0)),
            full((D, H)), full((D, H)), full((1, H)),
            full((D, H)), full((D, H)), full((1, H)),
            full((H, C)), full((H, C)),
        ],
        out_specs=[
            pl.BlockSpec((NC, TN, D), lambda i: (0, i, 0)),
            pl.BlockSpec((TN, 16), lambda i: (i, 0)),
            pl.BlockSpec((TN, C), lambda i: (i, 0)),
            pl.BlockSpec((TN, 8), lambda i: (i, 0)),
        ],
        out_shape=[
            jax.ShapeDtypeStruct((NC, N2, D), jnp.float32),
            jax.ShapeDtypeStruct((N2, 16), jnp.float32),
            jax.ShapeDtypeStruct((N2, C), jnp.float32),
            jax.ShapeDtypeStruct((N2, 8), jnp.float32),
        ],
    )(x2, aggx, degp, We1_r, We1_n, We1_b, Wa1_r, Wa1_n, Wa1_b, Wa2_r, Wa2_n)


def _tc2_body(z1_ref, aggz_ref, aggt_ref, r8_ref, invd_ref, batch_ref,
              we2r_ref, we2n_ref, we2b_ref, wa2b_ref,
              wc1_ref, bc1_ref, wc2_ref, bc2_ref,
              out_ref, pooled_ref):
    i = pl.program_id(0)
    invd = invd_ref[:, :1]
    z1a = z1_ref[0]
    z1b = z1_ref[1]
    mza = aggz_ref[0] * invd
    mzb = aggz_ref[1] * invd
    z2 = jnp.maximum(
        jnp.dot(z1a, we2r_ref[:D, :])
        + jnp.dot(z1b, we2r_ref[D:, :])
        + jnp.dot(mza, we2n_ref[:D, :])
        + jnp.dot(mzb, we2n_ref[D:, :])
        + we2b_ref[...], 0.0)
    meant = (aggt_ref[0, :, :C] + aggt_ref[1, :, :C]) * invd
    s2 = r8_ref[...] + meant + wa2b_ref[...]
    sm = jnp.exp(s2 - jnp.max(s2, axis=-1, keepdims=True))
    sm = sm / jnp.sum(sm, axis=-1, keepdims=True)
    bt = batch_ref[0, 0, :]
    onehot = (bt[:, None] == lax.broadcasted_iota(jnp.int32, (TN, B), 1)
              ).astype(jnp.float32)

    @pl.when(i == 0)
    def _():
        pooled_ref[...] = jnp.zeros((C * B, H), jnp.float32)

    for cc in range(C):
        w = onehot * sm[:, cc:cc + 1]
        pooled_ref[pl.ds(cc * B, B), :] += lax.dot_general(
            w, z2, dimension_numbers=(((0,), (0,)), ((), ())))

    @pl.when(i == NT - 1)
    def _():
        acc = jnp.zeros((B, H), jnp.float32)
        for cc in range(C):
            acc += jnp.dot(pooled_ref[pl.ds(cc * B, B), :],
                           wc1_ref[pl.ds(cc * H, H), :])
        h = jnp.maximum(acc + bc1_ref[...], 0.0)
        out_ref[...] = (jnp.dot(h, wc2_ref[...])
                        + bc2_ref[...]).reshape(1, B)


def _tc2(z1, aggz, aggt, r8, invd, batch3,
         We2_r, We2_n, We2_b, Wa2_b, Wc1, bc1, Wc2, bc2):
    full = lambda shape: pl.BlockSpec(shape, lambda i: (0,) * len(shape))
    return pl.pallas_call(
        _tc2_body,
        grid=(NT,),
        in_specs=[
            pl.BlockSpec((NC, TN, D), lambda i: (0, i, 0)),
            pl.BlockSpec((NC, TN, D), lambda i: (0, i, 0)),
            pl.BlockSpec((NC, TN, 16), lambda i: (0, i, 0)),
            pl.BlockSpec((TN, C), lambda i: (i, 0)),
            pl.BlockSpec((TN, 8), lambda i: (i, 0)),
            pl.BlockSpec((1, 1, TN), lambda i: (i, 0, 0)),
            full((H, H)), full((H, H)), full((1, H)), full((1, C)),
            full((C * H, H)), full((1, H)), full((H, 1)), full((1, 1)),
        ],
        out_specs=pl.BlockSpec((1, B), lambda i: (0, 0)),
        out_shape=jax.ShapeDtypeStruct((1, B), jnp.float32),
        scratch_shapes=[pltpu.VMEM((C * B, H), jnp.float32)],
    )(z1, aggz, aggt, r8, invd, batch3,
      We2_r, We2_n, We2_b, Wa2_b, Wc1, bc1, Wc2, bc2)


def kernel(x, edge_index, batch, We1_r, We1_n, We1_b, We2_r, We2_n, We2_b,
           Wa1_r, Wa1_n, Wa1_b, Wa2_r, Wa2_n, Wa2_b, Wc1, bc1, Wc2, bc2):
    f32 = jnp.float32
    x2 = jnp.zeros((N2, D), f32).at[:N].set(x)
    pad = N + jax.lax.iota(jnp.int32, EPAD - E) % (N2 - N)
    srcR = jnp.concatenate([edge_index[0].astype(jnp.int32), pad]
                           ).reshape(NROW, CH)
    dstR = jnp.concatenate([edge_index[1].astype(jnp.int32), pad]
                           ).reshape(NROW, CH)
    batch3 = jnp.full((N2,), B, jnp.int32).at[:N].set(
        batch.astype(jnp.int32)).reshape(NT, 1, TN)
    zrow = jnp.zeros((RPT, D), f32)
    zdeg = jnp.zeros((RPT, 16), f32)
    ones16 = jnp.ones((CH, 16), f32)

    degp = _sca(dstR, zdeg, ones16).reshape(2 * NC, N2, 16)
    aggx = _scb(x2, srcR, dstR, zrow)
    z1, t16, r8, invd = _tc1(x2, aggx, degp, We1_r, We1_n,
                             We1_b.reshape(1, H), Wa1_r, Wa1_n,
                             Wa1_b.reshape(1, H), Wa2_r, Wa2_n)
    aggz = _scc(z1.reshape(NC * N2, D), srcR, dstR, zrow)
    aggt = _scd(t16, srcR, dstR, zdeg)
    out = _tc2(z1, aggz, aggt, r8, invd, batch3,
               We2_r, We2_n, We2_b.reshape(1, H), Wa2_b.reshape(1, C),
               Wc1, bc1.reshape(1, H), Wc2, bc2.reshape(1, 1))
    return out[0]


# speedup vs baseline: 2.5930x; 1.0001x over previous
import functools

import jax
import jax.numpy as jnp
from jax import lax
from jax.experimental import pallas as pl
from jax.experimental.pallas import tpu as pltpu
from jax.experimental.pallas import tpu_sc as plsc

N = 10000
E = 320000
D = 128
H = 256
C = 8
B = 64

N2 = 10240
CH = 64
NB = 4
NG = 3
BLK = 16
NBT = 4
BLKT = 16
NC = 2
NS = 16
EPAD = 327680
NROW = EPAD // CH
RPT = N2 // NS
NCH1 = NROW // (NC * NS)
NCHZ = NROW // NS
TN = 512
NT = N2 // TN

@functools.lru_cache(maxsize=None)
def _mesh():
    return plsc.VectorSubcoreMesh(core_axis_name="c", subcore_axis_name="s",
                                  num_cores=NC, num_subcores=NS)


def _idx_block(src_hbm, dst_hbm, sed, ded, st, blkrow, blk, coff):
    pltpu.sync_copy(src_hbm.at[pl.ds(blkrow, blk)], sed.at[st])
    pltpu.sync_copy(dst_hbm.at[pl.ds(blkrow, blk)], ded.at[st])
    if coff is not None:
        for j in range(blk):
            for k in range(CH // 16):
                sl = pl.ds(k * 16, 16)
                sed[st, j, sl] = sed[st, j, sl] + coff


def _ring(table, src_hbm, dst_hbm, sed, ded, rows, gsems, ssems, acc,
          nb, ng, blk, base, nch, coff=None):
    nblk = nch // blk

    def ref(st, j, k):
        return (st, j - k) if j >= k else (1 - st, blk + j - k)

    def finalize(st1, j1, b):
        pltpu.make_async_copy(table.at[sed.at[st1, j1]], rows.at[b],
                              gsems[b]).wait()
        pltpu.async_copy(rows.at[b], acc.at[ded.at[st1, j1]], ssems[b],
                         add=True)

    def swait(st2, j2, b):
        pltpu.make_async_copy(rows.at[b], acc.at[ded.at[st2, j2]],
                              ssems[b]).wait()

    def block(st, blkrow, first):
        _idx_block(src_hbm, dst_hbm, sed, ded, st, blkrow, blk, coff)
        for j in range(blk):
            b = j % nb
            if not (first and j < ng):
                st1, j1 = ref(st, j, ng)
                finalize(st1, j1, (j - ng) % nb)
            if not (first and j < nb):
                st2, j2 = ref(st, j, nb)
                swait(st2, j2, b)
            pltpu.async_copy(table.at[sed.at[st, j]], rows.at[b], gsems[b])

    block(0, base, True)
    block(1, base + blk, False)

    def pair(p, carry):
        row0 = base + 2 * p * blk
        block(0, row0, False)
        block(1, row0 + blk, False)
        return carry

    lax.fori_loop(1, nblk // 2, pair, 0)
    for j in range(blk, blk + ng):
        finalize(1, j - ng, (j - ng) % nb)
    for j in range(blk - nb, blk):
        swait(1, j, j % nb)


def _sca_body(dst_hbm, zdeg_hbm, ones_hbm, degp_hbm, accd0, accd1, ded,
              ones_v, sem0, sem1):
    c = lax.axis_index("c")
    s = lax.axis_index("s")
    sems = (sem0, sem1)
    accs = (accd0, accd1)
    pltpu.sync_copy(zdeg_hbm, accd0.at[pl.ds(s * RPT, RPT)])
    pltpu.sync_copy(zdeg_hbm, accd1.at[pl.ds(s * RPT, RPT)])
    pltpu.sync_copy(ones_hbm, ones_v)
    plsc.subcore_barrier()

    base = (c * NS + s) * NCH1
    nblk = NCH1 // BLK
    na = 2

    def swait(st, j, b):
        pltpu.make_async_copy(ones_v, accs[b].at[ded.at[st, j]],
                              sems[b]).wait()

    def block(st, blkrow, first):
        pltpu.sync_copy(dst_hbm.at[pl.ds(blkrow, BLK)], ded.at[st])
        for j in range(BLK):
            b = j % na
            if first and j < na:
                pass
            elif j < na:
                swait(1 - st, BLK - na + j, b)
            else:
                swait(st, j - na, b)
            pltpu.async_copy(ones_v, accs[b].at[ded.at[st, j]], sems[b],
                             add=True)

    block(0, base, True)
    block(1, base + BLK, False)

    def pair(p, carry):
        row0 = base + 2 * p * BLK
        block(0, row0, False)
        block(1, row0 + BLK, False)
        return carry

    lax.fori_loop(1, nblk // 2, pair, 0)
    for j in range(BLK - na, BLK):
        swait(1, j, j % na)

    plsc.subcore_barrier()
    pltpu.sync_copy(accd0.at[pl.ds(s * RPT, RPT)],
                    degp_hbm.at[2 * c, pl.ds(s * RPT, RPT)])
    pltpu.sync_copy(accd1.at[pl.ds(s * RPT, RPT)],
                    degp_hbm.at[2 * c + 1, pl.ds(s * RPT, RPT)])


@functools.lru_cache(maxsize=None)
def _make_sca():
    return pl.kernel(
        _sca_body,
        out_type=jax.ShapeDtypeStruct((2 * NC, N2, 16), jnp.float32),
        mesh=_mesh(),
        compiler_params=pltpu.CompilerParams(use_tc_tiling_on_sc=False),
        scratch_types=[
            pltpu.VMEM_SHARED((N2, 16), jnp.float32),
            pltpu.VMEM_SHARED((N2, 16), jnp.float32),
            pltpu.VMEM((2, BLK, CH), jnp.int32),
            pltpu.VMEM((CH, 16), jnp.float32),
            pltpu.SemaphoreType.DMA,
            pltpu.SemaphoreType.DMA,
        ],
    )


def _sca(*args):
    return _make_sca()(*args)


def _scw_body(table_hbm, src_hbm, dst_hbm, zrow_hbm, agg_hbm,
              acc, sed, ded, rows, g0, g1, g2, g3, s0, s1, s2, s3,
              *, split32, use_coff):
    c = lax.axis_index("c")
    s = lax.axis_index("s")
    pltpu.sync_copy(zrow_hbm, acc.at[pl.ds(s * RPT, RPT)])
    plsc.subcore_barrier()

    if split32:
        base = (c * NS + s) * NCH1
        nch = NCH1
    else:
        base = s * NCHZ
        nch = NCHZ
    _ring(table_hbm, src_hbm, dst_hbm, sed, ded, rows, (g0, g1, g2, g3),
          (s0, s1, s2, s3), acc, NB, NG, BLK, base, nch,
          coff=(c * N2 if use_coff else None))

    plsc.subcore_barrier()
    pltpu.sync_copy(acc.at[pl.ds(s * RPT, RPT)],
                    agg_hbm.at[c, pl.ds(s * RPT, RPT)])


@functools.lru_cache(maxsize=None)
def _make_scw(split32, use_coff):
    body = functools.partial(_scw_body, split32=split32, use_coff=use_coff)
    return pl.kernel(
        body,
        out_type=jax.ShapeDtypeStruct((NC, N2, D), jnp.float32),
        mesh=_mesh(),
        compiler_params=pltpu.CompilerParams(use_tc_tiling_on_sc=False),
        scratch_types=[
            pltpu.VMEM_SHARED((N2, D), jnp.float32),
            pltpu.VMEM((2, BLK, CH), jnp.int32),
            pltpu.VMEM((2, BLK, CH), jnp.int32),
            pltpu.VMEM((NB, CH, D), jnp.float32),
        ] + [pltpu.SemaphoreType.DMA] * 8,
    )


def _scb(x2, srcR, dstR, zrow):
    return _make_scw(True, False)(x2, srcR, dstR, zrow)


def _scc(z1f, srcR, dstR, zrow):
    return _make_scw(False, True)(z1f, srcR, dstR, zrow)


def _scd_body(t16_hbm, src_hbm, dst_hbm, zdeg_hbm, aggt_hbm,
              acct, sed, ded, trows, g0, g1, g2, g3, s0, s1, s2, s3):
    c = lax.axis_index("c")
    s = lax.axis_index("s")
    pltpu.sync_copy(zdeg_hbm, acct.at[pl.ds(s * RPT, RPT)])
    plsc.subcore_barrier()

    _ring(t16_hbm, src_hbm, dst_hbm, sed, ded, trows, (g0, g1, g2, g3),
          (s0, s1, s2, s3), acct, NBT, NBT - 1, BLKT, (c * NS + s) * NCH1,
          NCH1)

    plsc.subcore_barrier()
    pltpu.sync_copy(acct.at[pl.ds(s * RPT, RPT)],
                    aggt_hbm.at[c, pl.ds(s * RPT, RPT)])


@functools.lru_cache(maxsize=None)
def _make_scd():
    return pl.kernel(
        _scd_body,
        out_type=jax.ShapeDtypeStruct((NC, N2, 16), jnp.float32),
        mesh=_mesh(),
        compiler_params=pltpu.CompilerParams(use_tc_tiling_on_sc=False),
        scratch_types=[
            pltpu.VMEM_SHARED((N2, 16), jnp.float32),
            pltpu.VMEM((2, BLKT, CH), jnp.int32),
            pltpu.VMEM((2, BLKT, CH), jnp.int32),
            pltpu.VMEM((NBT, CH, 16), jnp.float32),
        ] + [pltpu.SemaphoreType.DMA] * 8,
    )


def _scd(*args):
    return _make_scd()(*args)


def _tc1_body(x_ref, aggx_ref, degp_ref,
              we1r_ref, we1n_ref, we1b_ref, wa1r_ref, wa1n_ref, wa1b_ref,
              wa2r_ref, wa2n_ref,
              z1_ref, t16_ref, r8_ref, invd_ref):
    xt = x_ref[...]
    deg = (degp_ref[0, :, :1] + degp_ref[1, :, :1]
           + degp_ref[2, :, :1] + degp_ref[3, :, :1])
    invd = 1.0 / jnp.maximum(deg, 1.0)
    meanx = (aggx_ref[0] + aggx_ref[1]) * invd
    z1 = jnp.maximum(
        jnp.dot(xt, we1r_ref[...])
        + jnp.dot(meanx, we1n_ref[...]) + we1b_ref[...], 0.0)
    s1 = (jnp.dot(xt, wa1r_ref[...])
          + jnp.dot(meanx, wa1n_ref[...]) + wa1b_ref[...])
    z1_ref[0] = z1[:, :D]
    z1_ref[1] = z1[:, D:]
    t16_ref[...] = jnp.concatenate(
        [jnp.dot(s1, wa2n_ref[...]),
         jnp.zeros((TN, 16 - C), jnp.float32)], axis=1)
    r8_ref[...] = jnp.dot(s1, wa2r_ref[...])
    invd_ref[...] = jnp.broadcast_to(invd, (TN, 8))


def _tc1(x2, aggx, degp, We1_r, We1_n, We1_b, Wa1_r, Wa1_n, Wa1_b,
         Wa2_r, Wa2_n):
    full = lambda shape: pl.BlockSpec(shape, lambda i: (0,) * len(shape))
    return pl.pallas_call(
        _tc1_body,
        grid=(NT,),
        in_specs=[
            pl.BlockSpec((TN, D), lambda i: (i, 0)),
            pl.BlockSpec((NC, TN, D), lambda i: (0, i, 0)),
            pl.BlockSpec((2 * NC, TN, 16), lambda i: (0, i, 0)),
            full((D, H)), full((D, H)), full((1, H)),
            full((D, H)), full((D, H)), full((1, H)),
            full((H, C)), full((H, C)),
        ],
        out_specs=[
            pl.BlockSpec((NC, TN, D), lambda i: (0, i, 0)),
            pl.BlockSpec((TN, 16), lambda i: (i, 0)),
            pl.BlockSpec((TN, C), lambda i: (i, 0)),
            pl.BlockSpec((TN, 8), lambda i: (i, 0)),
        ],
        out_shape=[
            jax.ShapeDtypeStruct((NC, N2, D), jnp.float32),
            jax.ShapeDtypeStruct((N2, 16), jnp.float32),
            jax.ShapeDtypeStruct((N2, C), jnp.float32),
            jax.ShapeDtypeStruct((N2, 8), jnp.float32),
        ],
    )(x2, aggx, degp, We1_r, We1_n, We1_b, Wa1_r, Wa1_n, Wa1_b, Wa2_r, Wa2_n)


def _tc2_body(z1_ref, aggz_ref, aggt_ref, r8_ref, invd_ref, batch_ref,
              we2r_ref, we2n_ref, we2b_ref, wa2b_ref,
              wc1_ref, bc1_ref, wc2_ref, bc2_ref,
              out_ref, pooled_ref):
    i = pl.program_id(0)
    invd = invd_ref[:, :1]
    z1a = z1_ref[0]
    z1b = z1_ref[1]
    mza = aggz_ref[0] * invd
    mzb = aggz_ref[1] * invd
    z2 = jnp.maximum(
        jnp.dot(z1a, we2r_ref[:D, :])
        + jnp.dot(z1b, we2r_ref[D:, :])
        + jnp.dot(mza, we2n_ref[:D, :])
        + jnp.dot(mzb, we2n_ref[D:, :])
        + we2b_ref[...], 0.0)
    meant = (aggt_ref[0, :, :C] + aggt_ref[1, :, :C]) * invd
    s2 = r8_ref[...] + meant + wa2b_ref[...]
    sm = jnp.exp(s2 - jnp.max(s2, axis=-1, keepdims=True))
    sm = sm / jnp.sum(sm, axis=-1, keepdims=True)
    bt = batch_ref[0, 0, :]
    onehot = (bt[:, None] == lax.broadcasted_iota(jnp.int32, (TN, B), 1)
              ).astype(jnp.float32)

    @pl.when(i == 0)
    def _():
        pooled_ref[...] = jnp.zeros((C * B, H), jnp.float32)

    for cc in range(C):
        w = onehot * sm[:, cc:cc + 1]
        pooled_ref[pl.ds(cc * B, B), :] += lax.dot_general(
            w, z2, dimension_numbers=(((0,), (0,)), ((), ())))

    @pl.when(i == NT - 1)
    def _():
        acc = jnp.zeros((B, H), jnp.float32)
        for cc in range(C):
            acc += jnp.dot(pooled_ref[pl.ds(cc * B, B), :],
                           wc1_ref[pl.ds(cc * H, H), :])
        h = jnp.maximum(acc + bc1_ref[...], 0.0)
        out_ref[...] = (jnp.dot(h, wc2_ref[...])
                        + bc2_ref[...]).reshape(1, B)


def _tc2(z1, aggz, aggt, r8, invd, batch3,
         We2_r, We2_n, We2_b, Wa2_b, Wc1, bc1, Wc2, bc2):
    full = lambda shape: pl.BlockSpec(shape, lambda i: (0,) * len(shape))
    return pl.pallas_call(
        _tc2_body,
        grid=(NT,),
        in_specs=[
            pl.BlockSpec((NC, TN, D), lambda i: (0, i, 0)),
            pl.BlockSpec((NC, TN, D), lambda i: (0, i, 0)),
            pl.BlockSpec((NC, TN, 16), lambda i: (0, i, 0)),
            pl.BlockSpec((TN, C), lambda i: (i, 0)),
            pl.BlockSpec((TN, 8), lambda i: (i, 0)),
            pl.BlockSpec((1, 1, TN), lambda i: (i, 0, 0)),
            full((H, H)), full((H, H)), full((1, H)), full((1, C)),
            full((C * H, H)), full((1, H)), full((H, 1)), full((1, 1)),
        ],
        out_specs=pl.BlockSpec((1, B), lambda i: (0, 0)),
        out_shape=jax.ShapeDtypeStruct((1, B), jnp.float32),
        scratch_shapes=[pltpu.VMEM((C * B, H), jnp.float32)],
    )(z1, aggz, aggt, r8, invd, batch3,
      We2_r, We2_n, We2_b, Wa2_b, Wc1, bc1, Wc2, bc2)


def kernel(x, edge_index, batch, We1_r, We1_n, We1_b, We2_r, We2_n, We2_b,
           Wa1_r, Wa1_n, Wa1_b, Wa2_r, Wa2_n, Wa2_b, Wc1, bc1, Wc2, bc2):
    f32 = jnp.float32
    x2 = jnp.zeros((N2, D), f32).at[:N].set(x)
    pad = N + jax.lax.iota(jnp.int32, EPAD - E) % (N2 - N)
    srcR = jnp.concatenate([edge_index[0].astype(jnp.int32), pad]
                           ).reshape(NROW, CH)
    dstR = jnp.concatenate([edge_index[1].astype(jnp.int32), pad]
                           ).reshape(NROW, CH)
    batch3 = jnp.full((N2,), B, jnp.int32).at[:N].set(
        batch.astype(jnp.int32)).reshape(NT, 1, TN)
    zrow = jnp.zeros((RPT, D), f32)
    zdeg = jnp.zeros((RPT, 16), f32)
    ones16 = jnp.ones((CH, 16), f32)

    degp = _sca(dstR, zdeg, ones16)
    aggx = _scb(x2, srcR, dstR, zrow)
    z1, t16, r8, invd = _tc1(x2, aggx, degp, We1_r, We1_n,
                             We1_b.reshape(1, H), Wa1_r, Wa1_n,
                             Wa1_b.reshape(1, H), Wa2_r, Wa2_n)
    aggz = _scc(z1.reshape(NC * N2, D), srcR, dstR, zrow)
    aggt = _scd(t16, srcR, dstR, zdeg)
    out = _tc2(z1, aggz, aggt, r8, invd, batch3,
               We2_r, We2_n, We2_b.reshape(1, H), Wa2_b.reshape(1, C),
               Wc1, bc1.reshape(1, H), Wc2, bc2.reshape(1, 1))
    return out[0]
